# baseline (speedup 1.0000x reference)
import functools

import jax
import jax.numpy as jnp
from jax import lax
from jax.experimental import pallas as pl
from jax.experimental.pallas import tpu as pltpu
from jax.experimental.pallas import tpu_sc as plsc

NC = 2
NS = 16
NW = NC * NS


def _pad_rows(rows, n_pad_to):
    extra = n_pad_to - rows.shape[0]
    if extra == 0:
        return rows
    reps = -(-extra // rows.shape[0])
    return jnp.concatenate([rows, jnp.tile(rows, reps)[:extra]])


def _sc_body(kc, kz, H, W, xf, tab, out,
             csrc_v, cdst_v, zdst_v, b0, b1, b2, zb,
             sg0, sg1, sg2, ss0, ss1, ss2, semz):
    wid = lax.axis_index("s") * NC + lax.axis_index("c")

    pltpu.sync_copy(tab.at[0, wid], csrc_v)
    pltpu.sync_copy(tab.at[1, wid], cdst_v)
    pltpu.sync_copy(tab.at[2, wid], zdst_v)
    csv = [csrc_v[pl.ds(16 * c, 16)] for c in range(-(-kc // 16))]
    cdv = [cdst_v[pl.ds(16 * c, 16)] for c in range(-(-kc // 16))]
    zdv = [zdst_v[pl.ds(16 * c, 16)] for c in range(-(-kz // 16))]

    def _idx(vs, k):
        return vs[k // 16][k % 16]

    zeros16 = jnp.zeros((16,), jnp.float32)
    nw16 = W // 16
    HQ = H // 4

    def zero_fire(k, q):
        d = _idx(zdv, k)
        pltpu.async_copy(zb, out.at[pl.ds(d, 1), pl.ds(q * HQ, HQ)], semz)

    def zero_drain(k, q):
        d = _idx(zdv, k)
        pltpu.make_async_copy(
            zb, out.at[pl.ds(d, 1), pl.ds(q * HQ, HQ)], semz).wait()

    bufs = (b0, b1, b2)
    gsems = (sg0, sg1, sg2)
    ssems = (ss0, ss1, ss2)
    H2 = H // 2
    TC_ = 2 * kc
    TZ = 4 * kz

    def gather_fire(t):
        s = _idx(csv, t // 2)
        pltpu.async_copy(
            xf.at[pl.ds(s, 1), pl.ds((t % 2) * H2, H2)],
            bufs[t % 3], gsems[t % 3])

    def gather_wait(t):
        s = _idx(csv, t // 2)
        pltpu.make_async_copy(
            xf.at[pl.ds(s, 1), pl.ds((t % 2) * H2, H2)],
            bufs[t % 3], gsems[t % 3]).wait()

    def scatter_fire(t):
        d = _idx(cdv, t // 2)
        pltpu.async_copy(
            bufs[t % 3], out.at[pl.ds(d, 1), pl.ds((t % 2) * H2, H2)],
            ssems[t % 3])

    def scatter_wait(t):
        d = _idx(cdv, t // 2)
        pltpu.make_async_copy(
            bufs[t % 3], out.at[pl.ds(d, 1), pl.ds((t % 2) * H2, H2)],
            ssems[t % 3]).wait()

    def zfire_q(z):
        zero_fire(z // 4, z % 4)

    def zdrain_q(z):
        zero_drain(z // 4, z % 4)

    if TC_ > 0:
        gather_fire(0)

    def zfill(t, carry):
        zb[0, t // nw16, pl.ds((t % nw16) * 16, 16)] = zeros16
        return carry

    lax.fori_loop(0, HQ * nw16, zfill, 0)

    for t in range(TC_):
        if t >= 2:
            scatter_wait(t - 2)
        if t + 1 < TC_:
            gather_fire(t + 1)
        for z in (2 * t, 2 * t + 1):
            if z < TZ:
                zfire_q(z)
        for z in (2 * t - 8, 2 * t - 7):
            if 0 <= z < TZ:
                zdrain_q(z)
        gather_wait(t)
        scatter_fire(t)
    for t in range(max(TC_ - 2, 0), TC_):
        scatter_wait(t)
    for z in range(2 * TC_, TZ):
        zfire_q(z)
    for z in range(max(min(2 * TC_ - 8, TZ), 0), TZ):
        zdrain_q(z)


def kernel(x, in_channels, out_channels):
    B, C_in, H, W = x.shape
    C_out = out_channels.shape[0]

    jmax = max(C_in, C_out - C_in)
    io = jnp.arange(C_out, dtype=jnp.int32)
    m_in = jnp.pad((in_channels != 0).astype(jnp.int32), (0, C_out - C_in))
    m_out = (out_channels != 0).astype(jnp.int32)
    cs_in = jnp.cumsum(m_in) - 1
    cs_out = jnp.cumsum(m_out) - 1
    ranks3 = jnp.stack([cs_in, cs_out, io - cs_out - 1])
    masks3 = jnp.stack([m_in, m_out, 1 - m_out])
    j = jnp.arange(jmax, dtype=jnp.int32)
    sel = (ranks3[:, None, :] == j[None, :, None]) & (masks3[:, None, :] == 1)
    pos3 = (sel.astype(jnp.int32) * io[None, None, :]).sum(-1)

    b = jnp.arange(B, dtype=jnp.int32)
    strides3 = jnp.array([C_in, C_out, C_out], dtype=jnp.int32)
    planes3 = (b[None, :, None] * strides3[:, None, None]
               + pos3[:, None, :])

    kc = -(-(B * C_in) // NW)
    kz = -(-(B * (C_out - C_in)) // NW)
    kp = -(-max(kc, kz) // 128) * 128

    def worker_rows(rows, k):
        t = _pad_rows(rows, NW * k).reshape(NW, k)
        return jnp.pad(t, ((0, 0), (0, kp - k)))

    tab = jnp.stack([
        worker_rows(planes3[0, :, :C_in].reshape(-1), kc),
        worker_rows(planes3[1, :, :C_in].reshape(-1), kc),
        worker_rows(planes3[2, :, :C_out - C_in].reshape(-1), kz),
    ])

    xf = x.reshape(B * C_in, H, W)

    mesh = plsc.VectorSubcoreMesh(
        core_axis_name="c", subcore_axis_name="s",
        num_cores=NC, num_subcores=NS)
    body = functools.partial(_sc_body, kc, kz, H, W)
    out = pl.kernel(
        body,
        out_type=jax.ShapeDtypeStruct((B * C_out, H, W), jnp.float32),
        mesh=mesh,
        compiler_params=pltpu.CompilerParams(use_tc_tiling_on_sc=True),
        scratch_types=[
            pltpu.VMEM((kp,), jnp.int32),
            pltpu.VMEM((kp,), jnp.int32),
            pltpu.VMEM((kp,), jnp.int32),
            pltpu.VMEM((1, H // 2, W), jnp.float32),
            pltpu.VMEM((1, H // 2, W), jnp.float32),
            pltpu.VMEM((1, H // 2, W), jnp.float32),
            pltpu.VMEM((1, H // 4, W), jnp.float32),
            pltpu.SemaphoreType.DMA,
            pltpu.SemaphoreType.DMA,
            pltpu.SemaphoreType.DMA,
            pltpu.SemaphoreType.DMA,
            pltpu.SemaphoreType.DMA,
            pltpu.SemaphoreType.DMA,
            pltpu.SemaphoreType.DMA,
        ],
    )(xf, tab)
    return out.reshape(B, C_out, H, W)


# speedup vs baseline: 1.0281x; 1.0281x over previous
import functools

import jax
import jax.numpy as jnp
from jax import lax
from jax.experimental import pallas as pl
from jax.experimental.pallas import tpu as pltpu
from jax.experimental.pallas import tpu_sc as plsc

NC = 2
NS = 16
NW = NC * NS


def _pad_rows(rows, n_pad_to):
    extra = n_pad_to - rows.shape[0]
    if extra == 0:
        return rows
    reps = -(-extra // rows.shape[0])
    return jnp.concatenate([rows, jnp.tile(rows, reps)[:extra]])


def _sc_body(kc, kz, H, W, xf, tab, out,
             csrc_v, cdst_v, zdst_v, b0, b1, zb,
             sg0, sg1, ss0, ss1, semz):
    wid = lax.axis_index("s") * NC + lax.axis_index("c")

    pltpu.sync_copy(tab.at[0, wid], csrc_v)
    pltpu.sync_copy(tab.at[1, wid], cdst_v)
    pltpu.sync_copy(tab.at[2, wid], zdst_v)
    csv = [csrc_v[pl.ds(16 * c, 16)] for c in range(-(-kc // 16))]
    cdv = [cdst_v[pl.ds(16 * c, 16)] for c in range(-(-kc // 16))]
    zdv = [zdst_v[pl.ds(16 * c, 16)] for c in range(-(-kz // 16))]

    def _idx(vs, k):
        return vs[k // 16][k % 16]

    zeros16 = jnp.zeros((16,), jnp.float32)
    nw16 = W // 16
    HQ = H // 4

    def zero_fire(k, q):
        d = _idx(zdv, k)
        pltpu.async_copy(zb, out.at[pl.ds(d, 1), pl.ds(q * HQ, HQ)], semz)

    def zero_drain(k, q):
        d = _idx(zdv, k)
        pltpu.make_async_copy(
            zb, out.at[pl.ds(d, 1), pl.ds(q * HQ, HQ)], semz).wait()

    bufs = (b0, b1)
    gsems = (sg0, sg1)
    ssems = (ss0, ss1)

    def gather_fire(k):
        s = _idx(csv, k)
        pltpu.async_copy(xf.at[pl.ds(s, 1)], bufs[k % 2], gsems[k % 2])

    def gather_wait(k):
        s = _idx(csv, k)
        pltpu.make_async_copy(
            xf.at[pl.ds(s, 1)], bufs[k % 2], gsems[k % 2]).wait()

    def scatter_fire(k):
        d = _idx(cdv, k)
        pltpu.async_copy(bufs[k % 2], out.at[pl.ds(d, 1)], ssems[k % 2])

    def scatter_wait(k):
        d = _idx(cdv, k)
        pltpu.make_async_copy(
            bufs[k % 2], out.at[pl.ds(d, 1)], ssems[k % 2]).wait()

    if kc > 0:
        gather_fire(0)

    def zfill(t, carry):
        zb[0, t // nw16, pl.ds((t % nw16) * 16, 16)] = zeros16
        return carry

    lax.fori_loop(0, HQ * nw16, zfill, 0)

    for k in range(kc):
        if k >= 1:
            scatter_wait(k - 1)
        if k + 1 < kc:
            gather_fire(k + 1)
        if k < kz:
            for q in range(4):
                zero_fire(k, q)
        if 2 <= k and k - 2 < kz:
            for q in range(4):
                zero_drain(k - 2, q)
        gather_wait(k)
        scatter_fire(k)
    if kc > 0:
        scatter_wait(kc - 1)
    for k in range(kc, kz):
        for q in range(4):
            zero_fire(k, q)
    for k in range(max(min(kc - 2, kz), 0), kz):
        for q in range(4):
            zero_drain(k, q)


def kernel(x, in_channels, out_channels):
    B, C_in, H, W = x.shape
    C_out = out_channels.shape[0]

    jmax = max(C_in, C_out - C_in)
    io = jnp.arange(C_out, dtype=jnp.int32)
    m_in = jnp.pad((in_channels != 0).astype(jnp.int32), (0, C_out - C_in))
    m_out = (out_channels != 0).astype(jnp.int32)
    cs_in = jnp.cumsum(m_in) - 1
    cs_out = jnp.cumsum(m_out) - 1
    ranks3 = jnp.stack([cs_in, cs_out, io - cs_out - 1])
    masks3 = jnp.stack([m_in, m_out, 1 - m_out])
    j = jnp.arange(jmax, dtype=jnp.int32)
    sel = (ranks3[:, None, :] == j[None, :, None]) & (masks3[:, None, :] == 1)
    pos3 = (sel.astype(jnp.int32) * io[None, None, :]).sum(-1)

    b = jnp.arange(B, dtype=jnp.int32)
    strides3 = jnp.array([C_in, C_out, C_out], dtype=jnp.int32)
    planes3 = (b[None, :, None] * strides3[:, None, None]
               + pos3[:, None, :])

    kc = -(-(B * C_in) // NW)
    kz = -(-(B * (C_out - C_in)) // NW)
    kp = -(-max(kc, kz) // 128) * 128

    def worker_rows(rows, k):
        t = _pad_rows(rows, NW * k).reshape(NW, k)
        return jnp.pad(t, ((0, 0), (0, kp - k)))

    tab = jnp.stack([
        worker_rows(planes3[0, :, :C_in].reshape(-1), kc),
        worker_rows(planes3[1, :, :C_in].reshape(-1), kc),
        worker_rows(planes3[2, :, :C_out - C_in].reshape(-1), kz),
    ])

    xf = x.reshape(B * C_in, H, W)

    mesh = plsc.VectorSubcoreMesh(
        core_axis_name="c", subcore_axis_name="s",
        num_cores=NC, num_subcores=NS)
    body = functools.partial(_sc_body, kc, kz, H, W)
    out = pl.kernel(
        body,
        out_type=jax.ShapeDtypeStruct((B * C_out, H, W), jnp.float32),
        mesh=mesh,
        compiler_params=pltpu.CompilerParams(use_tc_tiling_on_sc=True),
        scratch_types=[
            pltpu.VMEM((kp,), jnp.int32),
            pltpu.VMEM((kp,), jnp.int32),
            pltpu.VMEM((kp,), jnp.int32),
            pltpu.VMEM((1, H, W), jnp.float32),
            pltpu.VMEM((1, H, W), jnp.float32),
            pltpu.VMEM((1, H // 4, W), jnp.float32),
            pltpu.SemaphoreType.DMA,
            pltpu.SemaphoreType.DMA,
            pltpu.SemaphoreType.DMA,
            pltpu.SemaphoreType.DMA,
            pltpu.SemaphoreType.DMA,
        ],
    )(xf, tab)
    return out.reshape(B, C_out, H, W)
